# baseline (speedup 1.0000x reference)
import jax
import jax.numpy as jnp
from jax.experimental import pallas as pl

N = 100000
B = 128


def _sage(x, src, dst, W_l, b_l, W_r):
    msgs = jnp.take(x, src, axis=0)
    agg = jax.ops.segment_sum(msgs, dst, num_segments=N)
    deg = jax.ops.segment_sum(jnp.ones((src.shape[0], 1), x.dtype), dst, num_segments=N)
    mean = agg / jnp.maximum(deg, 1.0)
    return mean @ W_l.T + b_l + x @ W_r.T


def _final_kernel(pooled_ref, wc_ref, bc_ref, out_ref):
    out_ref[...] = jnp.sum(pooled_ref[...] * wc_ref[...], axis=1, keepdims=True) + bc_ref[0, 0]


def kernel(x, edge_index, batch_index, W1_l, b1, W1_r, W2_l, b2, W2_r, W3_l, b3, W3_r, Wc, bc):
    src, dst = edge_index[0], edge_index[1]
    h = jax.nn.leaky_relu(_sage(x, src, dst, W1_l, b1, W1_r), negative_slope=0.01)
    h = jax.nn.leaky_relu(_sage(h, src, dst, W2_l, b2, W2_r), negative_slope=0.01)
    h = _sage(x=h, src=src, dst=dst, W_l=W3_l, b_l=b3, W_r=W3_r)
    sums = jax.ops.segment_sum(h, batch_index, num_segments=B)
    counts = jax.ops.segment_sum(jnp.ones((h.shape[0], 1), h.dtype), batch_index, num_segments=B)
    means = sums / jnp.maximum(counts, 1.0)
    pooled = jnp.concatenate([means, sums], axis=1)
    out = pl.pallas_call(
        _final_kernel,
        out_shape=jax.ShapeDtypeStruct((B, 1), jnp.float32),
    )(pooled, Wc, bc.reshape(1, 1))
    return out
